# baseline (speedup 1.0000x reference)
import functools

import jax
import jax.numpy as jnp
from jax import lax
from jax.experimental import pallas as pl
from jax.experimental.pallas import tpu as pltpu
from jax.experimental.pallas import tpu_sc as plsc

D_MODEL = 64
D_PAD = 128
D_VOCAB_ROWS = 1000000
N_TOKENS = 4096 * 200
NUM_WORKERS = 32
CHUNK = 128
B_PER_W = N_TOKENS // NUM_WORKERS
N_CHUNKS = B_PER_W // CHUNK

_mesh = plsc.VectorSubcoreMesh(core_axis_name="c", subcore_axis_name="s")


@functools.partial(
    pl.kernel,
    mesh=_mesh,
    compiler_params=pltpu.CompilerParams(use_tc_tiling_on_sc=False),
    out_type=jax.ShapeDtypeStruct((N_TOKENS, D_PAD), jnp.float32),
    scratch_types=[
        pltpu.VMEM((N_CHUNKS, CHUNK), jnp.int32),
        [pltpu.VMEM((CHUNK, D_MODEL), jnp.float32) for _ in range(8)],
        [pltpu.SemaphoreType.DMA for _ in range(8)],
        [pltpu.SemaphoreType.DMA for _ in range(8)],
    ],
)
def _embed_sc(tok_hbm, table_hbm, out_hbm, idx_v, rows, gsem, wsem):
    wid = lax.axis_index("s") * 2 + lax.axis_index("c")
    base = wid * B_PER_W
    pltpu.sync_copy(tok_hbm.at[wid], idx_v)

    def gather_start(j, b):
        pltpu.async_copy(table_hbm.at[idx_v.at[j]], rows[b], gsem[b])

    def gather_wait(b):
        pltpu.make_async_copy(table_hbm.at[idx_v.at[0]], rows[b], gsem[b]).wait()

    def write_start(j, b):
        pltpu.async_copy(rows[b],
                         out_hbm.at[pl.ds(base + j * CHUNK, CHUNK),
                                    pl.ds(0, D_MODEL)],
                         wsem[b])

    def write_wait(b):
        pltpu.make_async_copy(rows[b],
                              out_hbm.at[pl.ds(base, CHUNK), pl.ds(0, D_MODEL)],
                              wsem[b]).wait()

    NBUF, DEPTH = 8, 6
    PRO = NBUF - DEPTH
    assert (N_CHUNKS - DEPTH - PRO) % NBUF == 0
    for j in range(DEPTH):
        gather_start(j, j % NBUF)
    for j in range(PRO):
        gather_wait(j % NBUF)
        write_start(j, j % NBUF)
        gather_start(j + DEPTH, (j + DEPTH) % NBUF)

    def body(i, _):
        j0 = PRO + i * NBUF
        for k in range(NBUF):
            b = (PRO + k) % NBUF
            j = j0 + k
            gather_wait(b)
            write_start(j, b)
            nb = (b + DEPTH) % NBUF
            write_wait(nb)
            gather_start(j + DEPTH, nb)
        return 0

    lax.fori_loop(0, (N_CHUNKS - DEPTH - PRO) // NBUF, body, 0)

    for k in range(DEPTH):
        j = N_CHUNKS - DEPTH + k
        gather_wait(j % NBUF)
        write_start(j, j % NBUF)
    for b in range(NBUF):
        write_wait(b)


def kernel(tokens, embed_weight):
    tok2 = (tokens * 2).reshape(NUM_WORKERS, N_CHUNKS, CHUNK)
    tablep = jnp.concatenate([embed_weight, embed_weight], axis=1)
    table2 = tablep.reshape(2 * D_VOCAB_ROWS, D_MODEL)
    out = _embed_sc(tok2, table2)
    return out[:, :D_MODEL].reshape(4096, 200, D_MODEL)


# speedup vs baseline: 1.1787x; 1.1787x over previous
import functools

import jax
import jax.numpy as jnp
from jax import lax
from jax.experimental import pallas as pl
from jax.experimental.pallas import tpu as pltpu
from jax.experimental.pallas import tpu_sc as plsc

D_MODEL = 64
D_PAD = 128
D_VOCAB_ROWS = 1000000
N_TOKENS = 4096 * 200
NUM_WORKERS = 32
CHUNK = 128
B_PER_W = N_TOKENS // NUM_WORKERS
N_CHUNKS = B_PER_W // CHUNK

_mesh = plsc.VectorSubcoreMesh(core_axis_name="c", subcore_axis_name="s")


@functools.partial(
    pl.kernel,
    mesh=_mesh,
    compiler_params=pltpu.CompilerParams(use_tc_tiling_on_sc=False),
    out_type=jax.ShapeDtypeStruct((N_TOKENS, D_PAD), jnp.float32),
    scratch_types=[
        pltpu.VMEM((N_CHUNKS, CHUNK), jnp.int32),
        [pltpu.VMEM((CHUNK, D_MODEL), jnp.float32) for _ in range(8)],
        [pltpu.SemaphoreType.DMA for _ in range(8)],
        [pltpu.SemaphoreType.DMA for _ in range(8)],
    ],
)
def _embed_sc(tok_hbm, table_hbm, out_hbm, idx_v, rows, gsem, wsem):
    wid = lax.axis_index("s") * 2 + lax.axis_index("c")
    base = wid * B_PER_W
    pltpu.sync_copy(tok_hbm.at[wid], idx_v)

    def gather_start(j, b):
        pltpu.async_copy(table_hbm.at[idx_v.at[j]], rows[b], gsem[b])

    def gather_wait(b):
        pltpu.make_async_copy(table_hbm.at[idx_v.at[0]], rows[b], gsem[b]).wait()

    def write_start(j, b):
        pltpu.async_copy(rows[b],
                         out_hbm.at[pl.ds(base + j * CHUNK, CHUNK),
                                    pl.ds(0, D_MODEL)],
                         wsem[b])

    def write_wait(b):
        pltpu.make_async_copy(rows[b],
                              out_hbm.at[pl.ds(base, CHUNK), pl.ds(0, D_MODEL)],
                              wsem[b]).wait()

    NBUF, DEPTH = 8, 7
    PRO = NBUF - DEPTH
    assert (N_CHUNKS - DEPTH - PRO) % NBUF == 0
    for j in range(DEPTH):
        gather_start(j, j % NBUF)
    for j in range(PRO):
        gather_wait(j % NBUF)
        write_start(j, j % NBUF)
        gather_start(j + DEPTH, (j + DEPTH) % NBUF)

    def body(i, _):
        j0 = PRO + i * NBUF
        for k in range(NBUF):
            b = (PRO + k) % NBUF
            j = j0 + k
            gather_wait(b)
            write_start(j, b)
            nb = (b + DEPTH) % NBUF
            write_wait(nb)
            gather_start(j + DEPTH, nb)
        return 0

    lax.fori_loop(0, (N_CHUNKS - DEPTH - PRO) // NBUF, body, 0)

    for k in range(DEPTH):
        j = N_CHUNKS - DEPTH + k
        gather_wait(j % NBUF)
        write_start(j, j % NBUF)
    for b in range(NBUF):
        write_wait(b)


def kernel(tokens, embed_weight):
    tok2 = (tokens * 2).reshape(NUM_WORKERS, N_CHUNKS, CHUNK)
    tablep = jnp.pad(embed_weight, ((0, 0), (0, D_PAD - D_MODEL)))
    table2 = tablep.reshape(2 * D_VOCAB_ROWS, D_MODEL)
    out = _embed_sc(tok2, table2)
    return out[:, :D_MODEL].reshape(4096, 200, D_MODEL)
